# baseline (speedup 1.0000x reference)
import jax
import jax.numpy as jnp
from jax.experimental import pallas as pl
from jax.experimental.pallas import tpu as pltpu

_H = 4
_K = 5
_TAU = 0.25
_NEG = -1e30
_TILE = 8


def _top5_tile(tile):
    work = tile
    t1 = None
    t5 = None
    for k in range(_K):
        m = jnp.max(work, axis=-1, keepdims=True)
        if k == 0:
            t1 = m
        t5 = m
        if k < _K - 1:
            work = jnp.where(work == m, _NEG, work)
    e = jnp.exp((tile - t1) * (1.0 / _TAU))
    u = jnp.where(tile >= t5, e, 0.0)
    denom = jnp.sum(u, axis=-1, keepdims=True)
    return u / denom


def _build_sel(p_ref, h, s_ref):
    nt = s_ref.shape[0] // _TILE

    def body(i, carry):
        r = i * _TILE
        s_ref[pl.ds(r, _TILE), :] = _top5_tile(p_ref[h, pl.ds(r, _TILE), :])
        return carry

    jax.lax.fori_loop(0, nt, body, 0, unroll=2)


def _dot(x, y):
    return jax.lax.dot_general(
        x, y, (((1,), (0,)), ((), ())), preferred_element_type=jnp.float32
    )


def _scatter_accum(j, hw, p1_ref, s_ref, acc_ref, ws_ref):
    @pl.when(j == 0)
    def _():
        acc_ref[...] = jnp.zeros_like(acc_ref)
        ws_ref[...] = jnp.zeros_like(ws_ref)

    dh = hw.shape[1] // _H
    for h in range(_H):
        _build_sel(p1_ref, h, s_ref)
        s = s_ref[...]
        hwh = hw[:, h * dh:(h + 1) * dh]
        acc_ref[h] = acc_ref[h] + _dot(s.T, hwh)
        ws_ref[h:h + 1, :] = ws_ref[h:h + 1, :] + jnp.sum(s, axis=0, keepdims=True)


def _gather_heads(p2_ref, acc_ref, ws_ref, s_ref):
    outs = []
    for h in range(_H):
        wsum = jnp.maximum(ws_ref[h], 1e-6).reshape(-1, 1)
        anchor = acc_ref[h] / wsum
        _build_sel(p2_ref, h, s_ref)
        outs.append(_dot(s_ref[...], anchor))
    return jnp.concatenate(outs, axis=1)


def _k_first(x_ref, w_ref, p1_ref, acc_ref, ws_ref, s_ref):
    hw = _dot(x_ref[...], w_ref[...])
    _scatter_accum(pl.program_id(0), hw, p1_ref, s_ref, acc_ref, ws_ref)


def _k_mid(p2_ref, acc_in_ref, ws_in_ref, b_ref, w_ref, p1_ref, acc_ref, ws_ref, s_ref):
    h = _gather_heads(p2_ref, acc_in_ref, ws_in_ref, s_ref) + b_ref[...]
    h = jnp.where(h > 0, h, jnp.exp(jnp.minimum(h, 0.0)) - 1.0)
    hw = _dot(h, w_ref[...])
    _scatter_accum(pl.program_id(0), hw, p1_ref, s_ref, acc_ref, ws_ref)


def _k_last(p2_ref, acc_in_ref, ws_in_ref, b_ref, y_ref, s_ref):
    h = _gather_heads(p2_ref, acc_in_ref, ws_in_ref, s_ref) + b_ref[...]
    m = jnp.max(h, axis=-1, keepdims=True)
    z = h - m
    y_ref[...] = z - jnp.log(jnp.sum(jnp.exp(z), axis=-1, keepdims=True))


def _anchor_spec(dh):
    return pl.BlockSpec((_H, 256, dh), lambda j: (0, 0, 0))


def _ws_spec():
    return pl.BlockSpec((8, 256), lambda j: (0, 0))


def _params():
    return pltpu.CompilerParams(dimension_semantics=("arbitrary",))


def kernel(x, edge_index, W0, b0, P1_0, P2_0, W1, b1, P1_1, P2_1, W2, b2, P1_2, P2_2):
    del edge_index
    n, din = x.shape
    a = P1_0.shape[2]
    bsz = 400 if n % 400 == 0 else n
    nb = n // bsz
    f32 = jnp.float32

    hid = W0.shape[1]
    out = W2.shape[1]
    dh0 = hid // _H
    dh1 = W1.shape[1] // _H
    dh2 = out // _H

    pblk = lambda: pl.BlockSpec((_H, bsz, a), lambda j: (0, j, 0))
    sel_scratch = pltpu.VMEM((bsz, a), f32)

    acc0, ws0 = pl.pallas_call(
        _k_first,
        grid=(nb,),
        in_specs=[
            pl.BlockSpec((bsz, din), lambda j: (j, 0)),
            pl.BlockSpec((din, hid), lambda j: (0, 0)),
            pblk(),
        ],
        out_specs=[_anchor_spec(dh0), _ws_spec()],
        out_shape=[
            jax.ShapeDtypeStruct((_H, a, dh0), f32),
            jax.ShapeDtypeStruct((8, a), f32),
        ],
        scratch_shapes=[sel_scratch],
        compiler_params=_params(),
    )(x, W0, P1_0)

    def mid(p2, acc_in, ws_in, bvec, w, p1, dh_out):
        return pl.pallas_call(
            _k_mid,
            grid=(nb,),
            in_specs=[
                pblk(),
                _anchor_spec(acc_in.shape[2]),
                _ws_spec(),
                pl.BlockSpec((1, bvec.shape[0]), lambda j: (0, 0)),
                pl.BlockSpec(w.shape, lambda j: (0, 0)),
                pblk(),
            ],
            out_specs=[_anchor_spec(dh_out), _ws_spec()],
            out_shape=[
                jax.ShapeDtypeStruct((_H, a, dh_out), f32),
                jax.ShapeDtypeStruct((8, a), f32),
            ],
            scratch_shapes=[sel_scratch],
            compiler_params=_params(),
        )(p2, acc_in, ws_in, bvec.reshape(1, -1), w, p1)

    acc1, ws1 = mid(P2_0, acc0, ws0, b0, W1, P1_1, dh1)
    acc2, ws2 = mid(P2_1, acc1, ws1, b1, W2, P1_2, dh2)

    y = pl.pallas_call(
        _k_last,
        grid=(nb,),
        in_specs=[
            pblk(),
            _anchor_spec(dh2),
            _ws_spec(),
            pl.BlockSpec((1, out), lambda j: (0, 0)),
        ],
        out_specs=pl.BlockSpec((bsz, out), lambda j: (j, 0)),
        out_shape=jax.ShapeDtypeStruct((n, out), f32),
        scratch_shapes=[sel_scratch],
        compiler_params=_params(),
    )(P2_2, acc2, ws2, b2.reshape(1, -1))

    return y


# speedup vs baseline: 16.0372x; 16.0372x over previous
import jax
import jax.numpy as jnp
from jax.experimental import pallas as pl
from jax.experimental.pallas import tpu as pltpu

_H = 4
_K = 5
_TAU = 0.25
_NEG = -1e30


def _top5_sel(p):
    t1 = jnp.max(p, axis=-1, keepdims=True)
    t = t1
    for _ in range(_K - 1):
        t = jnp.max(jnp.where(p < t, p, _NEG), axis=-1, keepdims=True)
    e = jnp.exp((p - t1) * (1.0 / _TAU))
    u = jnp.where(p >= t, e, 0.0)
    denom = jnp.sum(u, axis=-1, keepdims=True)
    return u / denom


def _dot(x, y):
    return jax.lax.dot_general(
        x, y, (((1,), (0,)), ((), ())), preferred_element_type=jnp.float32
    )


def _scatter_accum(j, hw, p1_ref, acc_ref, ws_ref):
    @pl.when(j == 0)
    def _():
        acc_ref[...] = jnp.zeros_like(acc_ref)
        ws_ref[...] = jnp.zeros_like(ws_ref)

    dh = hw.shape[1] // _H
    for h in range(_H):
        s = _top5_sel(p1_ref[h])
        hwh = hw[:, h * dh:(h + 1) * dh]
        acc_ref[h] = acc_ref[h] + _dot(s.T, hwh)
        ws_ref[h:h + 1, :] = ws_ref[h:h + 1, :] + jnp.sum(s, axis=0, keepdims=True)


def _gather_heads(p2_ref, acc_ref, ws_ref):
    outs = []
    for h in range(_H):
        wsum = jnp.maximum(ws_ref[h], 1e-6).reshape(-1, 1)
        anchor = acc_ref[h] / wsum
        s2 = _top5_sel(p2_ref[h])
        outs.append(_dot(s2, anchor))
    return jnp.concatenate(outs, axis=1)


def _k_first(x_ref, w_ref, p1_ref, acc_ref, ws_ref):
    hw = _dot(x_ref[...], w_ref[...])
    _scatter_accum(pl.program_id(0), hw, p1_ref, acc_ref, ws_ref)


def _k_mid(p2_ref, acc_in_ref, ws_in_ref, b_ref, w_ref, p1_ref, acc_ref, ws_ref):
    h = _gather_heads(p2_ref, acc_in_ref, ws_in_ref) + b_ref[...]
    h = jnp.where(h > 0, h, jnp.exp(jnp.minimum(h, 0.0)) - 1.0)
    hw = _dot(h, w_ref[...])
    _scatter_accum(pl.program_id(0), hw, p1_ref, acc_ref, ws_ref)


def _k_last(p2_ref, acc_in_ref, ws_in_ref, b_ref, y_ref):
    h = _gather_heads(p2_ref, acc_in_ref, ws_in_ref) + b_ref[...]
    m = jnp.max(h, axis=-1, keepdims=True)
    z = h - m
    y_ref[...] = z - jnp.log(jnp.sum(jnp.exp(z), axis=-1, keepdims=True))


def _anchor_spec(dh):
    return pl.BlockSpec((_H, 256, dh), lambda j: (0, 0, 0))


def _ws_spec():
    return pl.BlockSpec((8, 256), lambda j: (0, 0))


def _params():
    return pltpu.CompilerParams(dimension_semantics=("arbitrary",))


def kernel(x, edge_index, W0, b0, P1_0, P2_0, W1, b1, P1_1, P2_1, W2, b2, P1_2, P2_2):
    del edge_index
    n, din = x.shape
    a = P1_0.shape[2]
    bsz = 400 if n % 400 == 0 else n
    nb = n // bsz
    f32 = jnp.float32

    hid = W0.shape[1]
    out = W2.shape[1]
    dh0 = hid // _H
    dh1 = W1.shape[1] // _H
    dh2 = out // _H

    pblk = lambda: pl.BlockSpec((_H, bsz, a), lambda j: (0, j, 0))

    acc0, ws0 = pl.pallas_call(
        _k_first,
        grid=(nb,),
        in_specs=[
            pl.BlockSpec((bsz, din), lambda j: (j, 0)),
            pl.BlockSpec((din, hid), lambda j: (0, 0)),
            pblk(),
        ],
        out_specs=[_anchor_spec(dh0), _ws_spec()],
        out_shape=[
            jax.ShapeDtypeStruct((_H, a, dh0), f32),
            jax.ShapeDtypeStruct((8, a), f32),
        ],
        compiler_params=_params(),
    )(x, W0, P1_0)

    def mid(p2, acc_in, ws_in, bvec, w, p1, dh_out):
        return pl.pallas_call(
            _k_mid,
            grid=(nb,),
            in_specs=[
                pblk(),
                _anchor_spec(acc_in.shape[2]),
                _ws_spec(),
                pl.BlockSpec((1, bvec.shape[0]), lambda j: (0, 0)),
                pl.BlockSpec(w.shape, lambda j: (0, 0)),
                pblk(),
            ],
            out_specs=[_anchor_spec(dh_out), _ws_spec()],
            out_shape=[
                jax.ShapeDtypeStruct((_H, a, dh_out), f32),
                jax.ShapeDtypeStruct((8, a), f32),
            ],
            compiler_params=_params(),
        )(p2, acc_in, ws_in, bvec.reshape(1, -1), w, p1)

    acc1, ws1 = mid(P2_0, acc0, ws0, b0, W1, P1_1, dh1)
    acc2, ws2 = mid(P2_1, acc1, ws1, b1, W2, P1_2, dh2)

    y = pl.pallas_call(
        _k_last,
        grid=(nb,),
        in_specs=[
            pblk(),
            _anchor_spec(dh2),
            _ws_spec(),
            pl.BlockSpec((1, out), lambda j: (0, 0)),
        ],
        out_specs=pl.BlockSpec((bsz, out), lambda j: (j, 0)),
        out_shape=jax.ShapeDtypeStruct((n, out), f32),
        compiler_params=_params(),
    )(P2_2, acc2, ws2, b2.reshape(1, -1))

    return y


# speedup vs baseline: 20.5138x; 1.2791x over previous
import jax
import jax.numpy as jnp
from jax.experimental import pallas as pl
from jax.experimental.pallas import tpu as pltpu

_H = 4
_K = 5
_TAU = 0.25
_NEG = -1e30


def _top5_sel(p):
    t1 = jnp.max(p, axis=-1, keepdims=True)
    t = t1
    for _ in range(_K - 1):
        t = jnp.max(jnp.where(p < t, p, _NEG), axis=-1, keepdims=True)
    e = jnp.exp((p - t1) * (1.0 / _TAU))
    u = jnp.where(p >= t, e, 0.0)
    denom = jnp.sum(u, axis=-1, keepdims=True)
    return u / denom


def _dot(x, y):
    return jax.lax.dot_general(
        x, y, (((1,), (0,)), ((), ())), preferred_element_type=jnp.float32
    )


def _scatter_accum(j, hw, p1_ref, acc_ref, ws_ref):
    @pl.when(j == 0)
    def _():
        acc_ref[...] = jnp.zeros_like(acc_ref)
        ws_ref[...] = jnp.zeros_like(ws_ref)

    dh = hw.shape[1] // _H
    hn, b, a = p1_ref.shape
    s_all = _top5_sel(p1_ref[...].reshape(hn * b, a))
    for h in range(_H):
        s = s_all[h * b:(h + 1) * b, :]
        hwh = hw[:, h * dh:(h + 1) * dh]
        acc_ref[h] = acc_ref[h] + _dot(s.T, hwh)
        ws_ref[h:h + 1, :] = ws_ref[h:h + 1, :] + jnp.sum(s, axis=0, keepdims=True)


def _gather_heads(p2_ref, acc_ref, ws_ref):
    hn, b, a = p2_ref.shape
    s_all = _top5_sel(p2_ref[...].reshape(hn * b, a))
    outs = []
    for h in range(_H):
        wsum = jnp.maximum(ws_ref[h], 1e-6).reshape(-1, 1)
        anchor = acc_ref[h] / wsum
        outs.append(_dot(s_all[h * b:(h + 1) * b, :], anchor))
    return jnp.concatenate(outs, axis=1)


def _k_first(x_ref, w_ref, p1_ref, acc_ref, ws_ref):
    hw = _dot(x_ref[...], w_ref[...])
    _scatter_accum(pl.program_id(0), hw, p1_ref, acc_ref, ws_ref)


def _k_mid(p2_ref, acc_in_ref, ws_in_ref, b_ref, w_ref, p1_ref, acc_ref, ws_ref):
    h = _gather_heads(p2_ref, acc_in_ref, ws_in_ref) + b_ref[...]
    h = jnp.where(h > 0, h, jnp.exp(jnp.minimum(h, 0.0)) - 1.0)
    hw = _dot(h, w_ref[...])
    _scatter_accum(pl.program_id(0), hw, p1_ref, acc_ref, ws_ref)


def _k_last(p2_ref, acc_in_ref, ws_in_ref, b_ref, y_ref):
    h = _gather_heads(p2_ref, acc_in_ref, ws_in_ref) + b_ref[...]
    m = jnp.max(h, axis=-1, keepdims=True)
    z = h - m
    y_ref[...] = z - jnp.log(jnp.sum(jnp.exp(z), axis=-1, keepdims=True))


def _anchor_spec(dh):
    return pl.BlockSpec((_H, 256, dh), lambda j: (0, 0, 0))


def _ws_spec():
    return pl.BlockSpec((8, 256), lambda j: (0, 0))


def _params():
    return pltpu.CompilerParams(dimension_semantics=("arbitrary",))


def kernel(x, edge_index, W0, b0, P1_0, P2_0, W1, b1, P1_1, P2_1, W2, b2, P1_2, P2_2):
    del edge_index
    n, din = x.shape
    a = P1_0.shape[2]
    bsz = 400 if n % 400 == 0 else n
    nb = n // bsz
    f32 = jnp.float32

    hid = W0.shape[1]
    out = W2.shape[1]
    dh0 = hid // _H
    dh1 = W1.shape[1] // _H
    dh2 = out // _H

    pblk = lambda: pl.BlockSpec((_H, bsz, a), lambda j: (0, j, 0))

    acc0, ws0 = pl.pallas_call(
        _k_first,
        grid=(nb,),
        in_specs=[
            pl.BlockSpec((bsz, din), lambda j: (j, 0)),
            pl.BlockSpec((din, hid), lambda j: (0, 0)),
            pblk(),
        ],
        out_specs=[_anchor_spec(dh0), _ws_spec()],
        out_shape=[
            jax.ShapeDtypeStruct((_H, a, dh0), f32),
            jax.ShapeDtypeStruct((8, a), f32),
        ],
        compiler_params=_params(),
    )(x, W0, P1_0)

    def mid(p2, acc_in, ws_in, bvec, w, p1, dh_out):
        return pl.pallas_call(
            _k_mid,
            grid=(nb,),
            in_specs=[
                pblk(),
                _anchor_spec(acc_in.shape[2]),
                _ws_spec(),
                pl.BlockSpec((1, bvec.shape[0]), lambda j: (0, 0)),
                pl.BlockSpec(w.shape, lambda j: (0, 0)),
                pblk(),
            ],
            out_specs=[_anchor_spec(dh_out), _ws_spec()],
            out_shape=[
                jax.ShapeDtypeStruct((_H, a, dh_out), f32),
                jax.ShapeDtypeStruct((8, a), f32),
            ],
            compiler_params=_params(),
        )(p2, acc_in, ws_in, bvec.reshape(1, -1), w, p1)

    acc1, ws1 = mid(P2_0, acc0, ws0, b0, W1, P1_1, dh1)
    acc2, ws2 = mid(P2_1, acc1, ws1, b1, W2, P1_2, dh2)

    y = pl.pallas_call(
        _k_last,
        grid=(nb,),
        in_specs=[
            pblk(),
            _anchor_spec(dh2),
            _ws_spec(),
            pl.BlockSpec((1, out), lambda j: (0, 0)),
        ],
        out_specs=pl.BlockSpec((bsz, out), lambda j: (j, 0)),
        out_shape=jax.ShapeDtypeStruct((n, out), f32),
        compiler_params=_params(),
    )(P2_2, acc2, ws2, b2.reshape(1, -1))

    return y


# speedup vs baseline: 23.0071x; 1.1215x over previous
import jax
import jax.numpy as jnp
from jax.experimental import pallas as pl
from jax.experimental.pallas import tpu as pltpu

_H = 4
_K = 5
_TAU = 0.25
_NEG = -1e30


def _top5_sel(p):
    t1 = jnp.max(p, axis=-1, keepdims=True)
    t = t1
    for _ in range(_K - 1):
        t = jnp.max(jnp.where(p < t, p, _NEG), axis=-1, keepdims=True)
    e = jnp.exp((p - t1) * (1.0 / _TAU))
    u = jnp.where(p >= t, e, 0.0)
    denom = jnp.sum(u, axis=-1, keepdims=True)
    return u / denom


def _dot(x, y):
    return jax.lax.dot_general(
        x, y, (((1,), (0,)), ((), ())), preferred_element_type=jnp.float32
    )


def _scatter_accum(j, hw, p1_ref, acc_ref, ws_ref):
    @pl.when(j == 0)
    def _():
        acc_ref[...] = jnp.zeros_like(acc_ref)
        ws_ref[...] = jnp.zeros_like(ws_ref)

    dh = hw.shape[1] // _H
    hn, b, a = p1_ref.shape
    s_all = _top5_sel(p1_ref[...].reshape(hn * b, a))
    for h in range(_H):
        s = s_all[h * b:(h + 1) * b, :]
        hwh = hw[:, h * dh:(h + 1) * dh]
        acc_ref[h] = acc_ref[h] + _dot(s.T, hwh)
        ws_ref[h:h + 1, :] = ws_ref[h:h + 1, :] + jnp.sum(s, axis=0, keepdims=True)


def _gather_heads(p2_ref, acc_ref, ws_ref):
    hn, b, a = p2_ref.shape
    s_all = _top5_sel(p2_ref[...].reshape(hn * b, a))
    outs = []
    for h in range(_H):
        wsum = jnp.maximum(ws_ref[h], 1e-6).reshape(-1, 1)
        anchor = acc_ref[h] / wsum
        outs.append(_dot(s_all[h * b:(h + 1) * b, :], anchor))
    return jnp.concatenate(outs, axis=1)


def _k_first(x_ref, w_ref, p1_ref, acc_ref, ws_ref):
    hw = _dot(x_ref[...], w_ref[...])
    _scatter_accum(pl.program_id(0), hw, p1_ref, acc_ref, ws_ref)


def _k_mid(p2_ref, acc_in_ref, ws_in_ref, b_ref, w_ref, p1_ref, acc_ref, ws_ref):
    h = _gather_heads(p2_ref, acc_in_ref, ws_in_ref) + b_ref[...]
    h = jnp.where(h > 0, h, jnp.exp(jnp.minimum(h, 0.0)) - 1.0)
    hw = _dot(h, w_ref[...])
    _scatter_accum(pl.program_id(0), hw, p1_ref, acc_ref, ws_ref)


def _k_last(p2_ref, acc_in_ref, ws_in_ref, b_ref, y_ref):
    h = _gather_heads(p2_ref, acc_in_ref, ws_in_ref) + b_ref[...]
    m = jnp.max(h, axis=-1, keepdims=True)
    z = h - m
    y_ref[...] = z - jnp.log(jnp.sum(jnp.exp(z), axis=-1, keepdims=True))


def _anchor_spec(dh):
    return pl.BlockSpec((_H, 256, dh), lambda j: (0, 0, 0))


def _ws_spec():
    return pl.BlockSpec((8, 256), lambda j: (0, 0))


def _params():
    return pltpu.CompilerParams(dimension_semantics=("arbitrary",))


def kernel(x, edge_index, W0, b0, P1_0, P2_0, W1, b1, P1_1, P2_1, W2, b2, P1_2, P2_2):
    del edge_index
    n, din = x.shape
    a = P1_0.shape[2]
    bsz = 1000 if n % 1000 == 0 else n
    nb = n // bsz
    f32 = jnp.float32

    hid = W0.shape[1]
    out = W2.shape[1]
    dh0 = hid // _H
    dh1 = W1.shape[1] // _H
    dh2 = out // _H

    pblk = lambda: pl.BlockSpec((_H, bsz, a), lambda j: (0, j, 0))

    acc0, ws0 = pl.pallas_call(
        _k_first,
        grid=(nb,),
        in_specs=[
            pl.BlockSpec((bsz, din), lambda j: (j, 0)),
            pl.BlockSpec((din, hid), lambda j: (0, 0)),
            pblk(),
        ],
        out_specs=[_anchor_spec(dh0), _ws_spec()],
        out_shape=[
            jax.ShapeDtypeStruct((_H, a, dh0), f32),
            jax.ShapeDtypeStruct((8, a), f32),
        ],
        compiler_params=_params(),
    )(x, W0, P1_0)

    def mid(p2, acc_in, ws_in, bvec, w, p1, dh_out):
        return pl.pallas_call(
            _k_mid,
            grid=(nb,),
            in_specs=[
                pblk(),
                _anchor_spec(acc_in.shape[2]),
                _ws_spec(),
                pl.BlockSpec((1, bvec.shape[0]), lambda j: (0, 0)),
                pl.BlockSpec(w.shape, lambda j: (0, 0)),
                pblk(),
            ],
            out_specs=[_anchor_spec(dh_out), _ws_spec()],
            out_shape=[
                jax.ShapeDtypeStruct((_H, a, dh_out), f32),
                jax.ShapeDtypeStruct((8, a), f32),
            ],
            compiler_params=_params(),
        )(p2, acc_in, ws_in, bvec.reshape(1, -1), w, p1)

    acc1, ws1 = mid(P2_0, acc0, ws0, b0, W1, P1_1, dh1)
    acc2, ws2 = mid(P2_1, acc1, ws1, b1, W2, P1_2, dh2)

    y = pl.pallas_call(
        _k_last,
        grid=(nb,),
        in_specs=[
            pblk(),
            _anchor_spec(dh2),
            _ws_spec(),
            pl.BlockSpec((1, out), lambda j: (0, 0)),
        ],
        out_specs=pl.BlockSpec((bsz, out), lambda j: (j, 0)),
        out_shape=jax.ShapeDtypeStruct((n, out), f32),
        compiler_params=_params(),
    )(P2_2, acc2, ws2, b2.reshape(1, -1))

    return y
